# baseline (speedup 1.0000x reference)
import jax
import jax.numpy as jnp
from jax.experimental import pallas as pl
from jax.experimental.pallas import tpu as pltpu

DIM = 1024
N_OPT = 17
TOK_BLK = 1024

_DN = (((1,), (1,)), ((), ()))


def _fused_mlp(z_ref, m_ref, w1_ref, b1_ref, w2_ref, b2_ref, o_ref, w1_bf,
               x_bf):
    @pl.when(pl.program_id(0) == 0)
    def _cast_weights():
        w1_bf[...] = w1_ref[...].astype(jnp.bfloat16)

    x_bf[:, :DIM] = z_ref[...].astype(jnp.bfloat16)
    x_bf[:, DIM:] = m_ref[...].astype(jnp.bfloat16)
    h = jax.lax.dot_general(x_bf[...], w1_bf[...], _DN,
                            preferred_element_type=jnp.float32)
    h = (h + b1_ref[...]).astype(jnp.bfloat16)
    h = (0.5 * h) * (1.0 + jax.lax.erf(h * 0.7071067811865476)).astype(
        jnp.bfloat16)
    out = jax.lax.dot_general(h, w2_ref[...].astype(jnp.bfloat16), _DN,
                              preferred_element_type=jnp.float32)
    o_ref[...] = out + b2_ref[...]


def kernel(z, m, W1, b1, W2, b2):
    n_tok = z.shape[0]
    b1r = b1.reshape(1, DIM)
    b2r = b2.reshape(1, N_OPT)

    grid = (n_tok // TOK_BLK,)
    return pl.pallas_call(
        _fused_mlp,
        grid=grid,
        in_specs=[
            pl.BlockSpec((TOK_BLK, DIM), lambda i: (i, 0)),
            pl.BlockSpec((TOK_BLK, DIM), lambda i: (i, 0)),
            pl.BlockSpec((DIM, 2 * DIM), lambda i: (0, 0)),
            pl.BlockSpec((1, DIM), lambda i: (0, 0)),
            pl.BlockSpec((N_OPT, DIM), lambda i: (0, 0)),
            pl.BlockSpec((1, N_OPT), lambda i: (0, 0)),
        ],
        out_specs=pl.BlockSpec((TOK_BLK, N_OPT), lambda i: (i, 0)),
        out_shape=jax.ShapeDtypeStruct((n_tok, N_OPT), jnp.float32),
        scratch_shapes=[pltpu.VMEM((DIM, 2 * DIM), jnp.bfloat16),
                        pltpu.VMEM((TOK_BLK, 2 * DIM), jnp.bfloat16)],
    )(z, m, W1, b1r, W2, b2r)
